# Initial kernel scaffold; baseline (speedup 1.0000x reference)
import jax
import jax.numpy as jnp
from jax.experimental import pallas as pl


def kernel(x):
    raise NotImplementedError("write your pallas kernel here")


# speedup vs baseline: 29.0631x; 29.0631x over previous
import jax
import jax.numpy as jnp
from jax.experimental import pallas as pl
from jax.experimental.pallas import tpu as pltpu


def _cross_pool_kernel(x0_ref, x1_ref, x2_ref, x3_ref, out_ref):
    x0 = x0_ref[0]
    x1 = x1_ref[0]
    x2 = x2_ref[0]
    x3 = x3_ref[0]

    b01 = x1 > x0
    b23 = x3 > x2
    w01 = jnp.where(b01, x1, x0)
    w23 = jnp.where(b23, x3, x2)
    bhi = w23 > w01

    b01m = b01[:, None, :]
    b23m = b23[:, None, :]
    bhim = bhi[:, None, :]
    x0v = x0[None, :, :]
    x1v = x1[None, :, :]
    x2v = x2[None, :, :]
    x3v = x3[None, :, :]

    lo = jnp.where(b01m, x1v, x0v)
    hi = jnp.where(b23m, x3v, x2v)
    out_ref[0] = jnp.where(bhim, hi, lo)


def kernel(x):
    B, C, H, W = x.shape
    k = 2
    Hp, Wp = H // k, W // k
    P = Hp * Wp

    x0 = x[:, :, 0::2, 0::2].reshape(B, C, P)
    x1 = x[:, :, 0::2, 1::2].reshape(B, C, P)
    x2 = x[:, :, 1::2, 0::2].reshape(B, C, P)
    x3 = x[:, :, 1::2, 1::2].reshape(B, C, P)

    pB = 256
    in_spec = pl.BlockSpec((1, C, pB), lambda b, p: (b, 0, p))
    return pl.pallas_call(
        _cross_pool_kernel,
        out_shape=jax.ShapeDtypeStruct((B, C, C, P), x.dtype),
        grid=(B, P // pB),
        in_specs=[in_spec, in_spec, in_spec, in_spec],
        out_specs=pl.BlockSpec((1, C, C, pB), lambda b, p: (b, 0, 0, p)),
        compiler_params=pltpu.CompilerParams(
            dimension_semantics=("parallel", "arbitrary"),
            vmem_limit_bytes=56 * 1024 * 1024,
        ),
        name="cross_pool_select",
    )(x0, x1, x2, x3)


# speedup vs baseline: 97.6894x; 3.3613x over previous
import jax
import jax.numpy as jnp
from jax.experimental import pallas as pl
from jax.experimental.pallas import tpu as pltpu

_LANE = 128


def _cross_pool_kernel(x_ref, out_ref):
    xin = x_ref[0]
    C = xin.shape[0]
    n_t = out_ref.shape[3] // _LANE

    lane = jax.lax.broadcasted_iota(jnp.int32, (1, _LANE), 1)
    wp = lane % 32
    q = lane // 32

    for t in range(n_t):
        srcs = [xin[:, (4 * t + s) * _LANE:(4 * t + s + 1) * _LANE]
                for s in range(4)]
        xj = []
        for dh, dw in ((0, 0), (0, 1), (1, 0), (1, 1)):
            idx = 64 * dh + 2 * wp + dw
            g = [jnp.take_along_axis(
                    srcs[s], jnp.broadcast_to(idx, srcs[s].shape), axis=-1)
                 for s in range(4)]
            xj.append(jnp.where(q < 2,
                                jnp.where(q == 0, g[0], g[1]),
                                jnp.where(q == 2, g[2], g[3])))
        x0, x1, x2, x3 = xj

        b01 = x1 > x0
        b23 = x3 > x2
        w01 = jnp.where(b01, x1, x0)
        w23 = jnp.where(b23, x3, x2)
        bhi = w23 > w01

        lo = jnp.where(b01[:, None, :], x1[None], x0[None])
        hi = jnp.where(b23[:, None, :], x3[None], x2[None])
        out_ref[0, :, :, t * _LANE:(t + 1) * _LANE] = jnp.where(
            bhi[:, None, :], hi, lo)


def kernel(x):
    B, C, H, W = x.shape
    Hp, Wp = H // 2, W // 2
    P = Hp * Wp

    x_flat = x.reshape(B, C, H * W)

    pB = 256
    return pl.pallas_call(
        _cross_pool_kernel,
        out_shape=jax.ShapeDtypeStruct((B, C, C, P), x.dtype),
        grid=(B, P // pB),
        in_specs=[pl.BlockSpec((1, C, 4 * pB), lambda b, p: (b, 0, p))],
        out_specs=pl.BlockSpec((1, C, C, pB), lambda b, p: (b, 0, 0, p)),
        compiler_params=pltpu.CompilerParams(
            dimension_semantics=("parallel", "arbitrary"),
            vmem_limit_bytes=56 * 1024 * 1024,
        ),
        name="cross_pool_select",
    )(x_flat)


# speedup vs baseline: 100.5845x; 1.0296x over previous
import jax
import jax.numpy as jnp
from jax.experimental import pallas as pl
from jax.experimental.pallas import tpu as pltpu

_LANE = 128


def _cross_pool_kernel(x_ref, out_ref):
    xin = x_ref[0]
    C = xin.shape[0]
    n_t = out_ref.shape[3] // _LANE

    lane = jax.lax.broadcasted_iota(jnp.int32, (1, _LANE), 1)
    wp = lane % 32
    q = lane // 32

    for t in range(n_t):
        srcs = [xin[:, (4 * t + s) * _LANE:(4 * t + s + 1) * _LANE]
                for s in range(4)]
        xj = []
        for dh, dw in ((0, 0), (0, 1), (1, 0), (1, 1)):
            idx = 64 * dh + 2 * wp + dw
            g = [jnp.take_along_axis(
                    srcs[s], jnp.broadcast_to(idx, srcs[s].shape), axis=-1)
                 for s in range(4)]
            xj.append(jnp.where(q < 2,
                                jnp.where(q == 0, g[0], g[1]),
                                jnp.where(q == 2, g[2], g[3])))
        x0, x1, x2, x3 = xj

        b01 = x1 > x0
        b23 = x3 > x2
        w01 = jnp.where(b01, x1, x0)
        w23 = jnp.where(b23, x3, x2)
        bhi = w23 > w01

        for cp0 in range(0, C, 4):
            csl = slice(cp0, cp0 + 4)
            lo = jnp.where(b01[csl][:, None, :], x1[None], x0[None])
            hi = jnp.where(b23[csl][:, None, :], x3[None], x2[None])
            out_ref[0, csl, :, t * _LANE:(t + 1) * _LANE] = jnp.where(
                bhi[csl][:, None, :], hi, lo)


def kernel(x):
    B, C, H, W = x.shape
    Hp, Wp = H // 2, W // 2
    P = Hp * Wp

    x_flat = x.reshape(B, C, H * W)

    pB = 256
    return pl.pallas_call(
        _cross_pool_kernel,
        out_shape=jax.ShapeDtypeStruct((B, C, C, P), x.dtype),
        grid=(B, P // pB),
        in_specs=[pl.BlockSpec((1, C, 4 * pB), lambda b, p: (b, 0, p))],
        out_specs=pl.BlockSpec((1, C, C, pB), lambda b, p: (b, 0, 0, p)),
        compiler_params=pltpu.CompilerParams(
            dimension_semantics=("parallel", "arbitrary"),
            vmem_limit_bytes=56 * 1024 * 1024,
        ),
        name="cross_pool_select",
    )(x_flat)
